# baseline (speedup 1.0000x reference)
import functools

import jax
import jax.numpy as jnp
from jax import lax
from jax.experimental import pallas as pl
from jax.experimental.pallas import tpu as pltpu
from jax.experimental.pallas import tpu_sc as plsc

B = 16384
D = 64
NC = 2
NS = 16
NW = NC * NS
PER_W = B // NW
CH = 128
NCHUNK = PER_W // CH

CB = 8192
NB = 68
HALF = CB * NB
N_ENT = 1100000


def _tc_transpose(ent_t):
    def body(a_ref, b_ref, o_ref):
        lane = lax.broadcasted_iota(jnp.int32, (D, D), 0)
        sub = lax.broadcasted_iota(jnp.int32, (D, D), 1)
        eye = (lane == sub).astype(jnp.float32)
        dn = (((0,), (0,)), ((), ()))
        o_ref[:, 0:64] = lax.dot_general(
            a_ref[...], eye, dn, precision=lax.Precision.HIGHEST)
        o_ref[:, 64:128] = lax.dot_general(
            b_ref[...], eye, dn, precision=lax.Precision.HIGHEST)

    return pl.pallas_call(
        body,
        grid=(NB,),
        in_specs=[
            pl.BlockSpec((D, CB), lambda b: (0, b)),
            pl.BlockSpec((D, CB), lambda b: (0, jnp.minimum(b + NB, 134))),
        ],
        out_specs=pl.BlockSpec((CB, 128), lambda b: (b, 0)),
        out_shape=jax.ShapeDtypeStruct((HALF, 128), jnp.float32),
        compiler_params=pltpu.CompilerParams(
            vmem_limit_bytes=63 * 1024 * 1024),
    )(ent_t, ent_t)


def _sc_scores(h, r, pos_t, neg_t, rel128, ent128):
    mesh = plsc.VectorSubcoreMesh(core_axis_name="c", subcore_axis_name="s")

    @functools.partial(
        pl.kernel,
        mesh=mesh,
        compiler_params=pltpu.CompilerParams(
            use_tc_tiling_on_sc=True, needs_layout_passes=False),
        out_type=(
            jax.ShapeDtypeStruct((B,), jnp.float32),
            jax.ShapeDtypeStruct((B,), jnp.float32),
        ),
        scratch_types=[
            pltpu.VMEM((CH,), jnp.int32),
            pltpu.VMEM((CH,), jnp.int32),
            pltpu.VMEM((CH,), jnp.int32),
            pltpu.VMEM((CH,), jnp.int32),
            pltpu.VMEM((CH,), jnp.int32),
            pltpu.VMEM((CH,), jnp.int32),
            pltpu.VMEM((CH,), jnp.int32),
            pltpu.VMEM((CH,), jnp.int32),
            pltpu.VMEM((CH,), jnp.int32),
            pltpu.VMEM((CH,), jnp.int32),
            pltpu.VMEM((CH,), jnp.int32),
            pltpu.VMEM((CH,), jnp.int32),
            pltpu.VMEM((CH,), jnp.int32),
            pltpu.VMEM((CH,), jnp.int32),
            pltpu.VMEM((CH, 128), jnp.float32),
            pltpu.VMEM((CH, 128), jnp.float32),
            pltpu.VMEM((CH, 128), jnp.float32),
            pltpu.VMEM((CH, 128), jnp.float32),
            pltpu.VMEM((CH, 128), jnp.float32),
            pltpu.VMEM((CH, 128), jnp.float32),
            pltpu.VMEM((64, 128), jnp.float32),
            pltpu.VMEM((CH,), jnp.float32),
            pltpu.VMEM((CH,), jnp.float32),
            pltpu.SemaphoreType.DMA,
            pltpu.SemaphoreType.DMA,
        ],
    )
    def body(h_hbm, r_hbm, pos_hbm, neg_hbm, rel_hbm, ent_hbm,
             pos_out, neg_out,
             hidx0, ridx0, pidx0, nidx0, hidx1, ridx1, pidx1, nidx1,
             hp0, pp0, np0, hp1, pp1, np1,
             hrow0, prow0, nrow0, hrow1, prow1, nrow1,
             relv, psc, nsc, sem0, sem1):
        wid = lax.axis_index("s") * NC + lax.axis_index("c")
        base = wid * PER_W
        sets = [
            (hidx0, ridx0, pidx0, nidx0, hp0, pp0, np0,
             hrow0, prow0, nrow0, sem0),
            (hidx1, ridx1, pidx1, nidx1, hp1, pp1, np1,
             hrow1, prow1, nrow1, sem1),
        ]

        pltpu.sync_copy(rel_hbm, relv)

        def stage(c, s):
            (hidx, ridx, pidx, nidx, hp, pp, np_,
             hrow, prow, nrow, sem) = sets[s]
            off = base + c * CH
            pltpu.sync_copy(h_hbm.at[pl.ds(off, CH)], hidx)
            pltpu.sync_copy(r_hbm.at[pl.ds(off, CH)], ridx)
            pltpu.sync_copy(pos_hbm.at[pl.ds(off, CH)], pidx)
            pltpu.sync_copy(neg_hbm.at[pl.ds(off, CH)], nidx)

            def pack(j, c2):
                sl = pl.ds(j * 16, 16)
                hv = hidx[sl]
                pv = pidx[sl]
                nv = nidx[sl]
                hp[sl] = jnp.where(hv < HALF, hv, hv - HALF)
                pp[sl] = jnp.where(pv < HALF, pv, pv - HALF)
                np_[sl] = jnp.where(nv < HALF, nv, nv - HALF)
                return c2

            lax.fori_loop(0, CH // 16, pack, 0)
            return (pltpu.async_copy(ent_hbm.at[hp], hrow, sem),
                    pltpu.async_copy(ent_hbm.at[pp], prow, sem),
                    pltpu.async_copy(ent_hbm.at[np_], nrow, sem))

        def compute(c, s, cps):
            (hidx, ridx, pidx, nidx, hp, pp, np_,
             hrow, prow, nrow, sem) = sets[s]
            for cp in cps:
                cp.wait()
            lane = lax.iota(jnp.int32, 16)

            def group(g, c2):
                sl = pl.ds(g * 16, 16)
                el = g * 16 + lane
                hb = jnp.where(hidx[sl] < HALF, 0, 64)
                pb = jnp.where(pidx[sl] < HALF, 0, 64)
                nb = jnp.where(nidx[sl] < HALF, 0, 64)
                rl = ridx[sl]

                def dim(d, accs):
                    accp, accn = accs
                    hv = plsc.load_gather(hrow, [el, hb + d])
                    rv = plsc.load_gather(relv, [rl, hb * 0 + d])
                    pv = plsc.load_gather(prow, [el, pb + d])
                    nv = plsc.load_gather(nrow, [el, nb + d])
                    hr = hv + rv
                    return (accp + hr * pv, accn + hr * nv)

                accp, accn = lax.fori_loop(
                    0, D, dim,
                    (jnp.zeros((16,), jnp.float32), jnp.zeros((16,), jnp.float32)))
                psc[sl] = accp
                nsc[sl] = accn
                return c2

            lax.fori_loop(0, CH // 16, group, 0)
            off = base + c * CH
            pltpu.sync_copy(psc, pos_out.at[pl.ds(off, CH)])
            pltpu.sync_copy(nsc, neg_out.at[pl.ds(off, CH)])

        cps = stage(0, 0)
        for c in range(NCHUNK):
            nxt = None
            if c + 1 < NCHUNK:
                nxt = stage(c + 1, (c + 1) % 2)
            compute(c, c % 2, cps)
            cps = nxt

    return body(h, r, pos_t, neg_t, rel128, ent128)


def _tc_loss(pos_s, neg_s):
    def body(p_ref, n_ref, o_ref):
        def lsig(x):
            return jnp.minimum(x, 0.0) - jnp.log1p(jnp.exp(-jnp.abs(x)))

        tot = jnp.sum(lsig(n_ref[...]) - lsig(p_ref[...]))
        o_ref[...] = (tot * (1.0 / B)).reshape(1, 1)

    return pl.pallas_call(
        body,
        out_shape=jax.ShapeDtypeStruct((1, 1), jnp.float32),
    )(pos_s.reshape(128, 128), neg_s.reshape(128, 128))


def kernel(h, r, pos_t, neg_t, relation_embed, entity_user_embed):
    ent128 = _tc_transpose(entity_user_embed.T)
    rel128 = jnp.concatenate([relation_embed, relation_embed], axis=1)
    pos_s, neg_s = _sc_scores(h.astype(jnp.int32), r.astype(jnp.int32),
                              pos_t.astype(jnp.int32), neg_t.astype(jnp.int32),
                              rel128, ent128)
    return _tc_loss(pos_s, neg_s).reshape(())


# speedup vs baseline: 1.8401x; 1.8401x over previous
import functools

import jax
import jax.numpy as jnp
from jax import lax
from jax.experimental import pallas as pl
from jax.experimental.pallas import tpu as pltpu
from jax.experimental.pallas import tpu_sc as plsc

B = 16384
D = 64
NC = 2
NS = 16
NW = NC * NS
PER_W = B // NW
CH = 128
NCHUNK = PER_W // CH

CB = 8192
NB = 68
HALF = CB * NB
N_ENT = 1100000


def _tc_transpose(ent_t):
    CBX = 5760

    def body(a_ref, b_ref, o_ref):
        lane = lax.broadcasted_iota(jnp.int32, (D, D), 0)
        sub = lax.broadcasted_iota(jnp.int32, (D, D), 1)
        eye = (lane == sub).astype(jnp.float32)
        dn = (((0,), (0,)), ((), ()))
        o_ref[CBX:, 0:64] = lax.dot_general(
            a_ref[:, CBX:], eye, dn, precision=lax.Precision.HIGHEST)
        o_ref[CBX:, 64:128] = lax.dot_general(
            b_ref[:, CBX:], eye, dn, precision=lax.Precision.HIGHEST)
        o_ref[0:CBX, 0:64] = jnp.swapaxes(a_ref[:, 0:CBX], 0, 1)
        o_ref[0:CBX, 64:128] = jnp.swapaxes(b_ref[:, 0:CBX], 0, 1)

    return pl.pallas_call(
        body,
        grid=(NB,),
        in_specs=[
            pl.BlockSpec((D, CB), lambda b: (0, b)),
            pl.BlockSpec((D, CB), lambda b: (0, jnp.minimum(b + NB, 134))),
        ],
        out_specs=pl.BlockSpec((CB, 128), lambda b: (b, 0)),
        out_shape=jax.ShapeDtypeStruct((HALF, 128), jnp.float32),
        compiler_params=pltpu.CompilerParams(
            vmem_limit_bytes=63 * 1024 * 1024),
    )(ent_t, ent_t)


def _sc_scores(h, r, pos_t, neg_t, rel128, ent128):
    mesh = plsc.VectorSubcoreMesh(core_axis_name="c", subcore_axis_name="s")

    @functools.partial(
        pl.kernel,
        mesh=mesh,
        compiler_params=pltpu.CompilerParams(
            use_tc_tiling_on_sc=True, needs_layout_passes=False),
        out_type=(
            jax.ShapeDtypeStruct((B,), jnp.float32),
            jax.ShapeDtypeStruct((B,), jnp.float32),
        ),
        scratch_types=[
            pltpu.VMEM((CH,), jnp.int32),
            pltpu.VMEM((CH,), jnp.int32),
            pltpu.VMEM((CH,), jnp.int32),
            pltpu.VMEM((CH,), jnp.int32),
            pltpu.VMEM((CH,), jnp.int32),
            pltpu.VMEM((CH,), jnp.int32),
            pltpu.VMEM((CH,), jnp.int32),
            pltpu.VMEM((CH,), jnp.int32),
            pltpu.VMEM((CH,), jnp.int32),
            pltpu.VMEM((CH,), jnp.int32),
            pltpu.VMEM((CH,), jnp.int32),
            pltpu.VMEM((CH,), jnp.int32),
            pltpu.VMEM((CH,), jnp.int32),
            pltpu.VMEM((CH,), jnp.int32),
            pltpu.VMEM((CH, 128), jnp.float32),
            pltpu.VMEM((CH, 128), jnp.float32),
            pltpu.VMEM((CH, 128), jnp.float32),
            pltpu.VMEM((CH, 128), jnp.float32),
            pltpu.VMEM((CH, 128), jnp.float32),
            pltpu.VMEM((CH, 128), jnp.float32),
            pltpu.VMEM((64, 128), jnp.float32),
            pltpu.VMEM((CH,), jnp.float32),
            pltpu.VMEM((CH,), jnp.float32),
            pltpu.SemaphoreType.DMA,
            pltpu.SemaphoreType.DMA,
        ],
    )
    def body(h_hbm, r_hbm, pos_hbm, neg_hbm, rel_hbm, ent_hbm,
             pos_out, neg_out,
             hidx0, ridx0, pidx0, nidx0, hidx1, ridx1, pidx1, nidx1,
             hp0, pp0, np0, hp1, pp1, np1,
             hrow0, prow0, nrow0, hrow1, prow1, nrow1,
             relv, psc, nsc, sem0, sem1):
        wid = lax.axis_index("s") * NC + lax.axis_index("c")
        base = wid * PER_W
        sets = [
            (hidx0, ridx0, pidx0, nidx0, hp0, pp0, np0,
             hrow0, prow0, nrow0, sem0),
            (hidx1, ridx1, pidx1, nidx1, hp1, pp1, np1,
             hrow1, prow1, nrow1, sem1),
        ]

        pltpu.sync_copy(rel_hbm, relv)

        def stage(c, s):
            (hidx, ridx, pidx, nidx, hp, pp, np_,
             hrow, prow, nrow, sem) = sets[s]
            off = base + c * CH
            pltpu.sync_copy(h_hbm.at[pl.ds(off, CH)], hidx)
            pltpu.sync_copy(r_hbm.at[pl.ds(off, CH)], ridx)
            pltpu.sync_copy(pos_hbm.at[pl.ds(off, CH)], pidx)
            pltpu.sync_copy(neg_hbm.at[pl.ds(off, CH)], nidx)

            def pack(j, c2):
                sl = pl.ds(j * 16, 16)
                hv = hidx[sl]
                pv = pidx[sl]
                nv = nidx[sl]
                hp[sl] = jnp.where(hv < HALF, hv, hv - HALF)
                pp[sl] = jnp.where(pv < HALF, pv, pv - HALF)
                np_[sl] = jnp.where(nv < HALF, nv, nv - HALF)
                return c2

            lax.fori_loop(0, CH // 16, pack, 0)
            return (pltpu.async_copy(ent_hbm.at[hp], hrow, sem),
                    pltpu.async_copy(ent_hbm.at[pp], prow, sem),
                    pltpu.async_copy(ent_hbm.at[np_], nrow, sem))

        def compute(c, s, cps):
            (hidx, ridx, pidx, nidx, hp, pp, np_,
             hrow, prow, nrow, sem) = sets[s]
            for cp in cps:
                cp.wait()
            lane = lax.iota(jnp.int32, 16)

            def group(g, c2):
                sl = pl.ds(g * 16, 16)
                el = g * 16 + lane
                hb = jnp.where(hidx[sl] < HALF, 0, 64)
                pb = jnp.where(pidx[sl] < HALF, 0, 64)
                nb = jnp.where(nidx[sl] < HALF, 0, 64)
                rl = ridx[sl]

                def dim(d, accs):
                    accp, accn = accs
                    hv = plsc.load_gather(hrow, [el, hb + d])
                    rv = plsc.load_gather(relv, [rl, hb * 0 + d])
                    pv = plsc.load_gather(prow, [el, pb + d])
                    nv = plsc.load_gather(nrow, [el, nb + d])
                    hr = hv + rv
                    return (accp + hr * pv, accn + hr * nv)

                accp, accn = lax.fori_loop(
                    0, D, dim,
                    (jnp.zeros((16,), jnp.float32), jnp.zeros((16,), jnp.float32)))
                psc[sl] = accp
                nsc[sl] = accn
                return c2

            lax.fori_loop(0, CH // 16, group, 0)
            off = base + c * CH
            pltpu.sync_copy(psc, pos_out.at[pl.ds(off, CH)])
            pltpu.sync_copy(nsc, neg_out.at[pl.ds(off, CH)])

        cps = stage(0, 0)
        for c in range(NCHUNK):
            nxt = None
            if c + 1 < NCHUNK:
                nxt = stage(c + 1, (c + 1) % 2)
            compute(c, c % 2, cps)
            cps = nxt

    return body(h, r, pos_t, neg_t, rel128, ent128)


def _tc_loss(pos_s, neg_s):
    def body(p_ref, n_ref, o_ref):
        def lsig(x):
            return jnp.minimum(x, 0.0) - jnp.log1p(jnp.exp(-jnp.abs(x)))

        tot = jnp.sum(lsig(n_ref[...]) - lsig(p_ref[...]))
        o_ref[...] = (tot * (1.0 / B)).reshape(1, 1)

    return pl.pallas_call(
        body,
        out_shape=jax.ShapeDtypeStruct((1, 1), jnp.float32),
    )(pos_s.reshape(128, 128), neg_s.reshape(128, 128))


def kernel(h, r, pos_t, neg_t, relation_embed, entity_user_embed):
    ent128 = _tc_transpose(entity_user_embed.T)
    rel128 = jnp.concatenate([relation_embed, relation_embed], axis=1)
    pos_s, neg_s = _sc_scores(h.astype(jnp.int32), r.astype(jnp.int32),
                              pos_t.astype(jnp.int32), neg_t.astype(jnp.int32),
                              rel128, ent128)
    return _tc_loss(pos_s, neg_s).reshape(())


# speedup vs baseline: 1.8442x; 1.0023x over previous
import functools

import jax
import jax.numpy as jnp
from jax import lax
from jax.experimental import pallas as pl
from jax.experimental.pallas import tpu as pltpu
from jax.experimental.pallas import tpu_sc as plsc

B = 16384
D = 64
NC = 2
NS = 16
NW = NC * NS
PER_W = B // NW
CH = 128
NCHUNK = PER_W // CH

CB = 8192
NB = 68
HALF = CB * NB
N_ENT = 1100000


def _tc_transpose(ent_t):
    def body(a_ref, b_ref, o_ref):
        o_ref[:, 0:64] = jnp.swapaxes(a_ref[...], 0, 1)
        o_ref[:, 64:128] = jnp.swapaxes(b_ref[...], 0, 1)

    return pl.pallas_call(
        body,
        grid=(NB,),
        in_specs=[
            pl.BlockSpec((D, CB), lambda b: (0, b)),
            pl.BlockSpec((D, CB), lambda b: (0, jnp.minimum(b + NB, 134))),
        ],
        out_specs=pl.BlockSpec((CB, 128), lambda b: (b, 0)),
        out_shape=jax.ShapeDtypeStruct((HALF, 128), jnp.float32),
        compiler_params=pltpu.CompilerParams(
            vmem_limit_bytes=63 * 1024 * 1024),
    )(ent_t, ent_t)


def _sc_scores(h, r, pos_t, neg_t, rel128, ent128):
    mesh = plsc.VectorSubcoreMesh(core_axis_name="c", subcore_axis_name="s")

    @functools.partial(
        pl.kernel,
        mesh=mesh,
        compiler_params=pltpu.CompilerParams(
            use_tc_tiling_on_sc=True, needs_layout_passes=False),
        out_type=(
            jax.ShapeDtypeStruct((B,), jnp.float32),
            jax.ShapeDtypeStruct((B,), jnp.float32),
        ),
        scratch_types=[
            pltpu.VMEM((CH,), jnp.int32),
            pltpu.VMEM((CH,), jnp.int32),
            pltpu.VMEM((CH,), jnp.int32),
            pltpu.VMEM((CH,), jnp.int32),
            pltpu.VMEM((CH,), jnp.int32),
            pltpu.VMEM((CH,), jnp.int32),
            pltpu.VMEM((CH,), jnp.int32),
            pltpu.VMEM((CH,), jnp.int32),
            pltpu.VMEM((CH,), jnp.int32),
            pltpu.VMEM((CH,), jnp.int32),
            pltpu.VMEM((CH,), jnp.int32),
            pltpu.VMEM((CH,), jnp.int32),
            pltpu.VMEM((CH,), jnp.int32),
            pltpu.VMEM((CH,), jnp.int32),
            pltpu.VMEM((CH, 128), jnp.float32),
            pltpu.VMEM((CH, 128), jnp.float32),
            pltpu.VMEM((CH, 128), jnp.float32),
            pltpu.VMEM((CH, 128), jnp.float32),
            pltpu.VMEM((CH, 128), jnp.float32),
            pltpu.VMEM((CH, 128), jnp.float32),
            pltpu.VMEM((64, 128), jnp.float32),
            pltpu.VMEM((CH,), jnp.float32),
            pltpu.VMEM((CH,), jnp.float32),
            pltpu.SemaphoreType.DMA,
            pltpu.SemaphoreType.DMA,
        ],
    )
    def body(h_hbm, r_hbm, pos_hbm, neg_hbm, rel_hbm, ent_hbm,
             pos_out, neg_out,
             hidx0, ridx0, pidx0, nidx0, hidx1, ridx1, pidx1, nidx1,
             hp0, pp0, np0, hp1, pp1, np1,
             hrow0, prow0, nrow0, hrow1, prow1, nrow1,
             relv, psc, nsc, sem0, sem1):
        wid = lax.axis_index("s") * NC + lax.axis_index("c")
        base = wid * PER_W
        sets = [
            (hidx0, ridx0, pidx0, nidx0, hp0, pp0, np0,
             hrow0, prow0, nrow0, sem0),
            (hidx1, ridx1, pidx1, nidx1, hp1, pp1, np1,
             hrow1, prow1, nrow1, sem1),
        ]

        pltpu.sync_copy(rel_hbm, relv)

        def stage(c, s):
            (hidx, ridx, pidx, nidx, hp, pp, np_,
             hrow, prow, nrow, sem) = sets[s]
            off = base + c * CH
            pltpu.sync_copy(h_hbm.at[pl.ds(off, CH)], hidx)
            pltpu.sync_copy(r_hbm.at[pl.ds(off, CH)], ridx)
            pltpu.sync_copy(pos_hbm.at[pl.ds(off, CH)], pidx)
            pltpu.sync_copy(neg_hbm.at[pl.ds(off, CH)], nidx)

            def pack(j, c2):
                sl = pl.ds(j * 16, 16)
                hv = hidx[sl]
                pv = pidx[sl]
                nv = nidx[sl]
                hp[sl] = jnp.where(hv < HALF, hv, hv - HALF)
                pp[sl] = jnp.where(pv < HALF, pv, pv - HALF)
                np_[sl] = jnp.where(nv < HALF, nv, nv - HALF)
                return c2

            lax.fori_loop(0, CH // 16, pack, 0)
            return (pltpu.async_copy(ent_hbm.at[hp], hrow, sem),
                    pltpu.async_copy(ent_hbm.at[pp], prow, sem),
                    pltpu.async_copy(ent_hbm.at[np_], nrow, sem))

        def compute(c, s, cps):
            (hidx, ridx, pidx, nidx, hp, pp, np_,
             hrow, prow, nrow, sem) = sets[s]
            for cp in cps:
                cp.wait()
            lane = lax.iota(jnp.int32, 16)

            def group(g, c2):
                sl = pl.ds(g * 16, 16)
                el = g * 16 + lane
                hb = jnp.where(hidx[sl] < HALF, 0, 64)
                pb = jnp.where(pidx[sl] < HALF, 0, 64)
                nb = jnp.where(nidx[sl] < HALF, 0, 64)
                rl = ridx[sl]

                def dim(d, accs):
                    accp, accn = accs
                    hv = plsc.load_gather(hrow, [el, hb + d])
                    rv = plsc.load_gather(relv, [rl, hb * 0 + d])
                    pv = plsc.load_gather(prow, [el, pb + d])
                    nv = plsc.load_gather(nrow, [el, nb + d])
                    hr = hv + rv
                    return (accp + hr * pv, accn + hr * nv)

                accp, accn = lax.fori_loop(
                    0, D, dim,
                    (jnp.zeros((16,), jnp.float32), jnp.zeros((16,), jnp.float32)))
                psc[sl] = accp
                nsc[sl] = accn
                return c2

            lax.fori_loop(0, CH // 16, group, 0)
            off = base + c * CH
            pltpu.sync_copy(psc, pos_out.at[pl.ds(off, CH)])
            pltpu.sync_copy(nsc, neg_out.at[pl.ds(off, CH)])

        cps = stage(0, 0)
        for c in range(NCHUNK):
            nxt = None
            if c + 1 < NCHUNK:
                nxt = stage(c + 1, (c + 1) % 2)
            compute(c, c % 2, cps)
            cps = nxt

    return body(h, r, pos_t, neg_t, rel128, ent128)


def _tc_loss(pos_s, neg_s):
    def body(p_ref, n_ref, o_ref):
        def lsig(x):
            return jnp.minimum(x, 0.0) - jnp.log1p(jnp.exp(-jnp.abs(x)))

        tot = jnp.sum(lsig(n_ref[...]) - lsig(p_ref[...]))
        o_ref[...] = (tot * (1.0 / B)).reshape(1, 1)

    return pl.pallas_call(
        body,
        out_shape=jax.ShapeDtypeStruct((1, 1), jnp.float32),
    )(pos_s.reshape(128, 128), neg_s.reshape(128, 128))


def kernel(h, r, pos_t, neg_t, relation_embed, entity_user_embed):
    ent128 = _tc_transpose(entity_user_embed.T)
    rel128 = jnp.concatenate([relation_embed, relation_embed], axis=1)
    pos_s, neg_s = _sc_scores(h.astype(jnp.int32), r.astype(jnp.int32),
                              pos_t.astype(jnp.int32), neg_t.astype(jnp.int32),
                              rel128, ent128)
    return _tc_loss(pos_s, neg_s).reshape(())


# speedup vs baseline: 1.9322x; 1.0477x over previous
import functools

import jax
import jax.numpy as jnp
from jax import lax
from jax.experimental import pallas as pl
from jax.experimental.pallas import tpu as pltpu
from jax.experimental.pallas import tpu_sc as plsc

B = 16384
D = 64
NC = 2
NS = 16
NW = NC * NS
PER_W = B // NW
CH = 128
NCHUNK = PER_W // CH

CB = 16384
NB = 34
HALF = CB * NB
N_ENT = 1100000


def _tc_transpose(ent_t):
    def body(a_ref, b_ref, o_ref):
        o_ref[:, 0:64] = jnp.swapaxes(a_ref[...], 0, 1)
        o_ref[:, 64:128] = jnp.swapaxes(b_ref[...], 0, 1)

    return pl.pallas_call(
        body,
        grid=(NB,),
        in_specs=[
            pl.BlockSpec((D, CB), lambda b: (0, b)),
            pl.BlockSpec((D, CB), lambda b: (0, b + NB)),
        ],
        out_specs=pl.BlockSpec((CB, 128), lambda b: (b, 0)),
        out_shape=jax.ShapeDtypeStruct((HALF, 128), jnp.float32),
        compiler_params=pltpu.CompilerParams(
            vmem_limit_bytes=110 * 1024 * 1024),
    )(ent_t, ent_t)


def _sc_scores(h, r, pos_t, neg_t, rel128, ent128):
    mesh = plsc.VectorSubcoreMesh(core_axis_name="c", subcore_axis_name="s")

    @functools.partial(
        pl.kernel,
        mesh=mesh,
        compiler_params=pltpu.CompilerParams(
            use_tc_tiling_on_sc=True, needs_layout_passes=False),
        out_type=(
            jax.ShapeDtypeStruct((B,), jnp.float32),
            jax.ShapeDtypeStruct((B,), jnp.float32),
        ),
        scratch_types=[
            pltpu.VMEM((CH,), jnp.int32),
            pltpu.VMEM((CH,), jnp.int32),
            pltpu.VMEM((CH,), jnp.int32),
            pltpu.VMEM((CH,), jnp.int32),
            pltpu.VMEM((CH,), jnp.int32),
            pltpu.VMEM((CH,), jnp.int32),
            pltpu.VMEM((CH,), jnp.int32),
            pltpu.VMEM((CH,), jnp.int32),
            pltpu.VMEM((CH,), jnp.int32),
            pltpu.VMEM((CH,), jnp.int32),
            pltpu.VMEM((CH,), jnp.int32),
            pltpu.VMEM((CH,), jnp.int32),
            pltpu.VMEM((CH,), jnp.int32),
            pltpu.VMEM((CH,), jnp.int32),
            pltpu.VMEM((CH, 128), jnp.float32),
            pltpu.VMEM((CH, 128), jnp.float32),
            pltpu.VMEM((CH, 128), jnp.float32),
            pltpu.VMEM((CH, 128), jnp.float32),
            pltpu.VMEM((CH, 128), jnp.float32),
            pltpu.VMEM((CH, 128), jnp.float32),
            pltpu.VMEM((64, 128), jnp.float32),
            pltpu.VMEM((CH,), jnp.float32),
            pltpu.VMEM((CH,), jnp.float32),
            pltpu.SemaphoreType.DMA,
            pltpu.SemaphoreType.DMA,
        ],
    )
    def body(h_hbm, r_hbm, pos_hbm, neg_hbm, rel_hbm, ent_hbm,
             pos_out, neg_out,
             hidx0, ridx0, pidx0, nidx0, hidx1, ridx1, pidx1, nidx1,
             hp0, pp0, np0, hp1, pp1, np1,
             hrow0, prow0, nrow0, hrow1, prow1, nrow1,
             relv, psc, nsc, sem0, sem1):
        wid = lax.axis_index("s") * NC + lax.axis_index("c")
        base = wid * PER_W
        sets = [
            (hidx0, ridx0, pidx0, nidx0, hp0, pp0, np0,
             hrow0, prow0, nrow0, sem0),
            (hidx1, ridx1, pidx1, nidx1, hp1, pp1, np1,
             hrow1, prow1, nrow1, sem1),
        ]

        pltpu.sync_copy(rel_hbm, relv)

        def stage(c, s):
            (hidx, ridx, pidx, nidx, hp, pp, np_,
             hrow, prow, nrow, sem) = sets[s]
            off = base + c * CH
            pltpu.sync_copy(h_hbm.at[pl.ds(off, CH)], hidx)
            pltpu.sync_copy(r_hbm.at[pl.ds(off, CH)], ridx)
            pltpu.sync_copy(pos_hbm.at[pl.ds(off, CH)], pidx)
            pltpu.sync_copy(neg_hbm.at[pl.ds(off, CH)], nidx)

            def pack(j, c2):
                sl = pl.ds(j * 16, 16)
                hv = hidx[sl]
                pv = pidx[sl]
                nv = nidx[sl]
                hp[sl] = jnp.where(hv < HALF, hv, hv - HALF)
                pp[sl] = jnp.where(pv < HALF, pv, pv - HALF)
                np_[sl] = jnp.where(nv < HALF, nv, nv - HALF)
                return c2

            lax.fori_loop(0, CH // 16, pack, 0)
            return (pltpu.async_copy(ent_hbm.at[hp], hrow, sem),
                    pltpu.async_copy(ent_hbm.at[pp], prow, sem),
                    pltpu.async_copy(ent_hbm.at[np_], nrow, sem))

        def compute(c, s, cps):
            (hidx, ridx, pidx, nidx, hp, pp, np_,
             hrow, prow, nrow, sem) = sets[s]
            for cp in cps:
                cp.wait()
            lane = lax.iota(jnp.int32, 16)

            def group(g, c2):
                sl = pl.ds(g * 16, 16)
                el = g * 16 + lane
                hb = jnp.where(hidx[sl] < HALF, 0, 64)
                pb = jnp.where(pidx[sl] < HALF, 0, 64)
                nb = jnp.where(nidx[sl] < HALF, 0, 64)
                rl = ridx[sl]

                def dim(d, accs):
                    accp, accn = accs
                    hv = plsc.load_gather(hrow, [el, hb + d])
                    rv = plsc.load_gather(relv, [rl, hb * 0 + d])
                    pv = plsc.load_gather(prow, [el, pb + d])
                    nv = plsc.load_gather(nrow, [el, nb + d])
                    hr = hv + rv
                    return (accp + hr * pv, accn + hr * nv)

                accp, accn = lax.fori_loop(
                    0, D, dim,
                    (jnp.zeros((16,), jnp.float32), jnp.zeros((16,), jnp.float32)))
                psc[sl] = accp
                nsc[sl] = accn
                return c2

            lax.fori_loop(0, CH // 16, group, 0)
            off = base + c * CH
            pltpu.sync_copy(psc, pos_out.at[pl.ds(off, CH)])
            pltpu.sync_copy(nsc, neg_out.at[pl.ds(off, CH)])

        cps = stage(0, 0)
        for c in range(NCHUNK):
            nxt = None
            if c + 1 < NCHUNK:
                nxt = stage(c + 1, (c + 1) % 2)
            compute(c, c % 2, cps)
            cps = nxt

    return body(h, r, pos_t, neg_t, rel128, ent128)


def _tc_loss(pos_s, neg_s):
    def body(p_ref, n_ref, o_ref):
        def lsig(x):
            return jnp.minimum(x, 0.0) - jnp.log1p(jnp.exp(-jnp.abs(x)))

        tot = jnp.sum(lsig(n_ref[...]) - lsig(p_ref[...]))
        o_ref[...] = (tot * (1.0 / B)).reshape(1, 1)

    return pl.pallas_call(
        body,
        out_shape=jax.ShapeDtypeStruct((1, 1), jnp.float32),
    )(pos_s.reshape(128, 128), neg_s.reshape(128, 128))


def kernel(h, r, pos_t, neg_t, relation_embed, entity_user_embed):
    ent128 = _tc_transpose(entity_user_embed.T)
    rel128 = jnp.concatenate([relation_embed, relation_embed], axis=1)
    pos_s, neg_s = _sc_scores(h.astype(jnp.int32), r.astype(jnp.int32),
                              pos_t.astype(jnp.int32), neg_t.astype(jnp.int32),
                              rel128, ent128)
    return _tc_loss(pos_s, neg_s).reshape(())
